# baseline (speedup 1.0000x reference)
import jax
import jax.numpy as jnp
from jax import lax
from jax.experimental import pallas as pl
from jax.experimental.pallas import tpu as pltpu
from jax.experimental.pallas import tpu_sc as plsc

_NSUB = 16
_NCORE = 2
_L = 128
_K = 4
_KG = 2
_STRIP = 16


def _rows_per_tile(n):
    return -(-(n // _NSUB) // 8) * 8


def _sc_agg_pair(h0, h1, src_flat, dst_flat, zeros_rows):
    n = h0.shape[0]
    nblk = src_flat.shape[0] // _L
    bpt = nblk // _NSUB
    rpt = _rows_per_tile(n)
    n_pad = rpt * _NSUB
    mesh = plsc.VectorSubcoreMesh(core_axis_name="c", subcore_axis_name="s")

    def body(h0_hbm, h1_hbm, src_hbm, dst_hbm, z_hbm, out_hbm,
             acc, src_v, dst_v, msg, sem):
        core = lax.axis_index("c")
        sub = lax.axis_index("s")
        r0 = sub * rpt
        pltpu.sync_copy(z_hbm, acc.at[pl.ds(r0, rpt)])
        plsc.subcore_barrier()

        def edge_loop(h_hbm):
            def blk(j, c):
                off = (sub + _NSUB * j) * _L
                pltpu.sync_copy(src_hbm.at[pl.ds(off, _L)], src_v)
                pltpu.sync_copy(dst_hbm.at[pl.ds(off, _L)], dst_v)
                pltpu.async_copy(h_hbm.at[src_v], msg, sem).wait()
                pltpu.sync_copy(msg, acc.at[dst_v], add=True)
                return c

            lax.fori_loop(0, bpt, blk, 0)

        @pl.when(core == 0)
        def _():
            edge_loop(h0_hbm)

        @pl.when(core == 1)
        def _():
            edge_loop(h1_hbm)

        plsc.subcore_barrier()

        @pl.when(core == 0)
        def _():
            pltpu.sync_copy(acc.at[pl.ds(r0, rpt)],
                            out_hbm.at[0, pl.ds(r0, rpt)])

        @pl.when(core == 1)
        def _():
            pltpu.sync_copy(acc.at[pl.ds(r0, rpt)],
                            out_hbm.at[1, pl.ds(r0, rpt)])

    scratch = [
        pltpu.VMEM_SHARED((n_pad, _L), jnp.float32),
        pltpu.VMEM((_L,), jnp.int32),
        pltpu.VMEM((_L,), jnp.int32),
        pltpu.VMEM((_L, _L), jnp.float32),
        pltpu.SemaphoreType.DMA,
    ]
    f = pl.kernel(
        body,
        out_type=jax.ShapeDtypeStruct((2, n_pad, _L), jnp.float32),
        mesh=mesh,
        scratch_types=scratch,
    )
    return f(h0, h1, src_flat, dst_flat, zeros_rows)


def _sc_count(dst_flat, ones_blk, zeros_rows, n):
    nblk = dst_flat.shape[0] // _L
    nw = _NSUB * _NCORE
    bpw = nblk // nw
    rpt = _rows_per_tile(n)
    n_pad = rpt * _NSUB
    mesh = plsc.VectorSubcoreMesh(core_axis_name="c", subcore_axis_name="s")

    def body(dst_hbm, ones_hbm, z_hbm, out_hbm, acc, dst_v, ones_v):
        core = lax.axis_index("c")
        sub = lax.axis_index("s")
        wid = sub * _NCORE + core
        r0 = sub * rpt
        pltpu.sync_copy(z_hbm, acc.at[pl.ds(r0, rpt)])
        pltpu.sync_copy(ones_hbm, ones_v)
        plsc.subcore_barrier()

        def blk(j, c):
            b = wid + nw * j
            pltpu.sync_copy(dst_hbm.at[pl.ds(b * _L, _L)], dst_v)
            pltpu.sync_copy(ones_v, acc.at[dst_v], add=True)
            return c

        lax.fori_loop(0, bpw, blk, 0)
        plsc.subcore_barrier()

        @pl.when(core == 0)
        def _():
            pltpu.sync_copy(acc.at[pl.ds(r0, rpt)],
                            out_hbm.at[0, pl.ds(r0, rpt)])

        @pl.when(core == 1)
        def _():
            pltpu.sync_copy(acc.at[pl.ds(r0, rpt)],
                            out_hbm.at[1, pl.ds(r0, rpt)])

    scratch = [
        pltpu.VMEM_SHARED((n_pad, _L), jnp.float32),
        pltpu.VMEM((_L,), jnp.int32),
        pltpu.VMEM((_L, _L), jnp.float32),
    ]
    f = pl.kernel(
        body,
        out_type=jax.ShapeDtypeStruct((2, n_pad, _L), jnp.float32),
        mesh=mesh,
        scratch_types=scratch,
    )
    return f(dst_flat, ones_blk, zeros_rows)


def _tc_layer(acc_list, h_chunks, cnt, Wl, Wr, bias, chunked_out, bn=1000):
    n = h_chunks.shape[1]
    grid_n = n // bn
    n_acc = len(acc_list)
    nc_h = h_chunks.shape[0]
    d_in = Wl.shape[0]
    d_out = Wl.shape[1]
    prec = jax.lax.Precision.HIGHEST

    def body(*refs):
        acc_refs = refs[:n_acc]
        h_ref, cnt_ref, wl_ref, wr_ref, b_ref, o_ref = refs[n_acc:]
        cr = cnt_ref[...]
        tot = cr[0, :, 0:1] + cr[1, :, 0:1]
        inv = 1.0 / jnp.maximum(tot, 1.0)
        wl = wl_ref[...]
        wr = wr_ref[...]
        tmp = jnp.zeros((bn, d_out), jnp.float32)
        ci = 0
        for ar in acc_refs:
            a = ar[...]
            for k in range(a.shape[0]):
                tmp = tmp + lax.dot(a[k], wl[ci * 128:(ci + 1) * 128, :],
                                    precision=prec)
                ci += 1
        tmp = tmp * inv
        h = h_ref[...]
        for k in range(nc_h):
            tmp = tmp + lax.dot(h[k], wr[k * 128:(k + 1) * 128, :],
                                precision=prec)
        out = jnp.maximum(tmp + b_ref[...], 0.0)
        if chunked_out:
            for k in range(d_out // 128):
                o_ref[k] = out[:, k * 128:(k + 1) * 128]
        else:
            o_ref[...] = out

    in_specs = (
        [pl.BlockSpec((2, bn, 128), lambda i: (0, i, 0)) for _ in acc_list]
        + [
            pl.BlockSpec((nc_h, bn, 128), lambda i: (0, i, 0)),
            pl.BlockSpec((2, bn, 128), lambda i: (0, i, 0)),
            pl.BlockSpec((d_in, d_out), lambda i: (0, 0)),
            pl.BlockSpec((d_in, d_out), lambda i: (0, 0)),
            pl.BlockSpec((1, d_out), lambda i: (0, 0)),
        ]
    )
    if chunked_out:
        out_spec = pl.BlockSpec((d_out // 128, bn, 128), lambda i: (0, i, 0))
        out_shape = jax.ShapeDtypeStruct((d_out // 128, n, 128), jnp.float32)
    else:
        out_spec = pl.BlockSpec((bn, d_out), lambda i: (i, 0))
        out_shape = jax.ShapeDtypeStruct((n, d_out), jnp.float32)

    return pl.pallas_call(
        body,
        grid=(grid_n,),
        in_specs=in_specs,
        out_specs=out_spec,
        out_shape=out_shape,
    )(*acc_list, h_chunks, cnt, Wl, Wr, bias)


def kernel(x, edge_index, Wl1, Wr1, b1, Wl2, Wr2, b2, Wl3, Wr3, b3):
    n, d_in = x.shape
    e = edge_index.shape[1]
    src = edge_index[0]
    dst = edge_index[1]
    f32 = jnp.float32

    nblk = -(-e // _L)
    blk_align = _NSUB * _NCORE
    nblk_pad = -(-nblk // blk_align) * blk_align
    epad = nblk_pad * _L - e
    n_pad0 = _rows_per_tile(n) * _NSUB
    assert n_pad0 - n >= 1
    pad_dst = n + jnp.arange(epad, dtype=jnp.int32) % jnp.int32(n_pad0 - n)
    src_p = jnp.concatenate([src, jnp.zeros((epad,), jnp.int32)])
    dst_p = jnp.concatenate([dst, pad_dst])

    rpt = _rows_per_tile(n)
    xc = x.reshape(n, d_in // 128, 128).transpose(1, 0, 2)
    zrows = jnp.zeros((rpt, _L), f32)
    ones_blk = jnp.ones((_L, _L), f32)

    cnt = _sc_count(dst_p, ones_blk, zrows, n)

    acc1 = _sc_agg_pair(xc[0], xc[1], src_p, dst_p, zrows)
    h1 = _tc_layer([acc1], xc, cnt, Wl1, Wr1, b1.reshape(1, -1), True)

    acc2a = _sc_agg_pair(h1[0], h1[1], src_p, dst_p, zrows)
    acc2b = _sc_agg_pair(h1[2], h1[3], src_p, dst_p, zrows)
    h2 = _tc_layer([acc2a, acc2b], h1, cnt, Wl2, Wr2, b2.reshape(1, -1), True)

    acc3a = _sc_agg_pair(h2[0], h2[1], src_p, dst_p, zrows)
    acc3b = _sc_agg_pair(h2[2], h2[3], src_p, dst_p, zrows)
    h3 = _tc_layer([acc3a, acc3b], h2, cnt, Wl3, Wr3, b3.reshape(1, -1), False)
    return h3


# speedup vs baseline: 1.3952x; 1.3952x over previous
import jax
import jax.numpy as jnp
from jax import lax
from jax.experimental import pallas as pl
from jax.experimental.pallas import tpu as pltpu
from jax.experimental.pallas import tpu_sc as plsc

_NSUB = 16
_NCORE = 2
_L = 128


def _sc_agg_pair(h0, h1, src, dst, zeros_rows):
    n = h0.shape[0]
    e = src.shape[0]
    nblk = e // _L
    rpt = -(-(n // _NSUB) // 8) * 8
    n_pad = rpt * _NSUB
    bpt = -(-nblk // _NSUB)
    mesh = plsc.VectorSubcoreMesh(core_axis_name="c", subcore_axis_name="s")

    def body(h0_hbm, h1_hbm, src_hbm, dst_hbm, z_hbm, out_hbm,
             acc, src_v, dst_v, msg, sem):
        core = lax.axis_index("c")
        sub = lax.axis_index("s")
        r0 = sub * rpt
        pltpu.sync_copy(z_hbm, acc.at[pl.ds(r0, rpt)])
        plsc.subcore_barrier()

        def edge_loop(h_hbm):
            def blk(j, carry):
                b = sub + _NSUB * j

                @pl.when(b < nblk)
                def _():
                    off = b * _L
                    pltpu.sync_copy(src_hbm.at[pl.ds(off, _L)], src_v)
                    pltpu.sync_copy(dst_hbm.at[pl.ds(off, _L)], dst_v)
                    pltpu.async_copy(h_hbm.at[src_v], msg, sem).wait()
                    pltpu.sync_copy(msg, acc.at[dst_v], add=True)

                return carry

            lax.fori_loop(0, bpt, blk, 0)

        @pl.when(core == 0)
        def _():
            edge_loop(h0_hbm)

        @pl.when(core == 1)
        def _():
            edge_loop(h1_hbm)

        plsc.subcore_barrier()

        @pl.when(core == 0)
        def _():
            pltpu.sync_copy(acc.at[pl.ds(r0, rpt)],
                            out_hbm.at[0, pl.ds(r0, rpt)])

        @pl.when(core == 1)
        def _():
            pltpu.sync_copy(acc.at[pl.ds(r0, rpt)],
                            out_hbm.at[1, pl.ds(r0, rpt)])

    f = pl.kernel(
        body,
        out_type=jax.ShapeDtypeStruct((2, n_pad, _L), jnp.float32),
        mesh=mesh,
        scratch_types=[
            pltpu.VMEM_SHARED((n_pad, _L), jnp.float32),
            pltpu.VMEM((_L,), jnp.int32),
            pltpu.VMEM((_L,), jnp.int32),
            pltpu.VMEM((_L, _L), jnp.float32),
            pltpu.SemaphoreType.DMA,
        ],
    )
    return f(h0, h1, src, dst, zeros_rows)


def _sc_count(dst, ones_blk, zeros_cnt, n):
    e = dst.shape[0]
    nblk = e // _L
    nw = _NSUB * _NCORE
    bpw = -(-nblk // nw)
    rpt = -(-(n // _NSUB) // 8) * 8
    n_pad = rpt * _NSUB
    mesh = plsc.VectorSubcoreMesh(core_axis_name="c", subcore_axis_name="s")

    def body(dst_hbm, ones_hbm, z_hbm, out_hbm, acc, dst_v, ones_v):
        core = lax.axis_index("c")
        sub = lax.axis_index("s")
        wid = sub * _NCORE + core
        r0 = sub * rpt
        pltpu.sync_copy(z_hbm, acc.at[pl.ds(r0, rpt)])
        pltpu.sync_copy(ones_hbm, ones_v)
        plsc.subcore_barrier()

        def blk(j, carry):
            b = wid + nw * j

            @pl.when(b < nblk)
            def _():
                pltpu.sync_copy(dst_hbm.at[pl.ds(b * _L, _L)], dst_v)
                pltpu.sync_copy(ones_v, acc.at[dst_v], add=True)

            return carry

        lax.fori_loop(0, bpw, blk, 0)
        plsc.subcore_barrier()

        @pl.when(core == 0)
        def _():
            pltpu.sync_copy(acc.at[pl.ds(r0, rpt)],
                            out_hbm.at[0, pl.ds(r0, rpt)])

        @pl.when(core == 1)
        def _():
            pltpu.sync_copy(acc.at[pl.ds(r0, rpt)],
                            out_hbm.at[1, pl.ds(r0, rpt)])

    f = pl.kernel(
        body,
        out_type=jax.ShapeDtypeStruct((2, n_pad, _L), jnp.float32),
        mesh=mesh,
        scratch_types=[
            pltpu.VMEM_SHARED((n_pad, _L), jnp.float32),
            pltpu.VMEM((_L,), jnp.int32),
            pltpu.VMEM((_L, _L), jnp.float32),
        ],
    )
    return f(dst, ones_blk, zeros_cnt)


def _tc_layer(acc_list, h_chunks, cnt, Wl, Wr, bias, chunked_out, bn=1000):
    n = h_chunks.shape[1]
    grid_n = n // bn
    n_acc = len(acc_list)
    nc_h = h_chunks.shape[0]
    d_in = Wl.shape[0]
    d_out = Wl.shape[1]
    prec = jax.lax.Precision.HIGHEST

    def body(*refs):
        acc_refs = refs[:n_acc]
        h_ref, cnt_ref, wl_ref, wr_ref, b_ref, o_ref = refs[n_acc:]
        cr = cnt_ref[...]
        tot = cr[0, :, 0:1] + cr[1, :, 0:1]
        inv = 1.0 / jnp.maximum(tot, 1.0)
        wl = wl_ref[...]
        wr = wr_ref[...]
        tmp = jnp.zeros((bn, d_out), jnp.float32)
        ci = 0
        for ar in acc_refs:
            a = ar[...]
            for k in range(a.shape[0]):
                tmp = tmp + lax.dot(a[k], wl[ci * 128:(ci + 1) * 128, :],
                                    precision=prec)
                ci += 1
        tmp = tmp * inv
        h = h_ref[...]
        for k in range(nc_h):
            tmp = tmp + lax.dot(h[k], wr[k * 128:(k + 1) * 128, :],
                                precision=prec)
        out = jnp.maximum(tmp + b_ref[...], 0.0)
        if chunked_out:
            for k in range(d_out // 128):
                o_ref[k] = out[:, k * 128:(k + 1) * 128]
        else:
            o_ref[...] = out

    in_specs = (
        [pl.BlockSpec((2, bn, 128), lambda i: (0, i, 0)) for _ in acc_list]
        + [
            pl.BlockSpec((nc_h, bn, 128), lambda i: (0, i, 0)),
            pl.BlockSpec((2, bn, 128), lambda i: (0, i, 0)),
            pl.BlockSpec((d_in, d_out), lambda i: (0, 0)),
            pl.BlockSpec((d_in, d_out), lambda i: (0, 0)),
            pl.BlockSpec((1, d_out), lambda i: (0, 0)),
        ]
    )
    if chunked_out:
        out_spec = pl.BlockSpec((d_out // 128, bn, 128), lambda i: (0, i, 0))
        out_shape = jax.ShapeDtypeStruct((d_out // 128, n, 128), jnp.float32)
    else:
        out_spec = pl.BlockSpec((bn, d_out), lambda i: (i, 0))
        out_shape = jax.ShapeDtypeStruct((n, d_out), jnp.float32)

    return pl.pallas_call(
        body,
        grid=(grid_n,),
        in_specs=in_specs,
        out_specs=out_spec,
        out_shape=out_shape,
    )(*acc_list, h_chunks, cnt, Wl, Wr, bias)


def kernel(x, edge_index, Wl1, Wr1, b1, Wl2, Wr2, b2, Wl3, Wr3, b3):
    n, d_in = x.shape
    e = edge_index.shape[1]
    src = edge_index[0]
    dst = edge_index[1]
    f32 = jnp.float32

    rpt = -(-(n // _NSUB) // 8) * 8
    xc = x.reshape(n, d_in // 128, 128).transpose(1, 0, 2)
    zrows = jnp.zeros((rpt, _L), f32)
    ones_blk = jnp.ones((_L, _L), f32)

    cnt = _sc_count(dst, ones_blk, zrows, n)

    acc1 = _sc_agg_pair(xc[0], xc[1], src, dst, zrows)
    h1 = _tc_layer([acc1], xc, cnt, Wl1, Wr1, b1.reshape(1, -1), True)

    acc2a = _sc_agg_pair(h1[0], h1[1], src, dst, zrows)
    acc2b = _sc_agg_pair(h1[2], h1[3], src, dst, zrows)
    h2 = _tc_layer([acc2a, acc2b], h1, cnt, Wl2, Wr2, b2.reshape(1, -1), True)

    acc3a = _sc_agg_pair(h2[0], h2[1], src, dst, zrows)
    acc3b = _sc_agg_pair(h2[2], h2[3], src, dst, zrows)
    h3 = _tc_layer([acc3a, acc3b], h2, cnt, Wl3, Wr3, b3.reshape(1, -1), False)
    return h3


# speedup vs baseline: 1.8377x; 1.3172x over previous
import jax
import jax.numpy as jnp
from jax import lax
from jax.experimental import pallas as pl
from jax.experimental.pallas import tpu as pltpu
from jax.experimental.pallas import tpu_sc as plsc

_NSUB = 16
_NCORE = 2
_L = 128


def _sc_agg_pair(h0, h1, src, dst, zeros_rows):
    n = h0.shape[0]
    e = src.shape[0]
    nblk = e // _L
    rpt = -(-(n // _NSUB) // 8) * 8
    n_pad = rpt * _NSUB
    bpt = -(-nblk // _NSUB)
    mesh = plsc.VectorSubcoreMesh(core_axis_name="c", subcore_axis_name="s")

    bpt2 = -(-bpt // 2)

    def body(h0_hbm, h1_hbm, src_hbm, dst_hbm, z_hbm, out_hbm,
             acc, src_v0, dst_v0, msg0, src_v1, dst_v1, msg1,
             gsem0, gsem1, ssem0, ssem1, isem0, isem1, isem2, isem3):
        core = lax.axis_index("c")
        sub = lax.axis_index("s")
        r0 = sub * rpt
        pltpu.sync_copy(z_hbm, acc.at[pl.ds(r0, rpt)])
        plsc.subcore_barrier()

        slots = ((src_v0, dst_v0, msg0, gsem0, ssem0, isem0, isem1),
                 (src_v1, dst_v1, msg1, gsem1, ssem1, isem2, isem3))

        def edge_loop(h_hbm):
            def one_slot(i2, p):
                src_v, dst_v, msg, gsem, ssem, ia, ib = slots[p]
                j = 2 * i2 + p
                b = sub + _NSUB * j

                @pl.when(i2 > 0)
                def _():
                    pltpu.make_async_copy(msg, acc.at[dst_v], ssem).wait()

                @pl.when(b < nblk)
                def _():
                    off = b * _L
                    d1 = pltpu.async_copy(src_hbm.at[pl.ds(off, _L)],
                                          src_v, ia)
                    d2 = pltpu.async_copy(dst_hbm.at[pl.ds(off, _L)],
                                          dst_v, ib)
                    d1.wait()
                    d2.wait()
                    pltpu.async_copy(h_hbm.at[src_v], msg, gsem).wait()
                    pltpu.async_copy(msg, acc.at[dst_v], ssem, add=True)

            def blk2(i2, carry):
                one_slot(i2, 0)
                one_slot(i2, 1)
                return carry

            lax.fori_loop(0, bpt2, blk2, 0)
            for p in range(2):
                src_v, dst_v, msg, gsem, ssem, ia, ib = slots[p]
                last_b = sub + _NSUB * (2 * (bpt2 - 1) + p)

                @pl.when(last_b < nblk)
                def _():
                    pltpu.make_async_copy(msg, acc.at[dst_v], ssem).wait()

        @pl.when(core == 0)
        def _():
            edge_loop(h0_hbm)

        @pl.when(core == 1)
        def _():
            edge_loop(h1_hbm)

        plsc.subcore_barrier()

        @pl.when(core == 0)
        def _():
            pltpu.sync_copy(acc.at[pl.ds(r0, rpt)],
                            out_hbm.at[0, pl.ds(r0, rpt)])

        @pl.when(core == 1)
        def _():
            pltpu.sync_copy(acc.at[pl.ds(r0, rpt)],
                            out_hbm.at[1, pl.ds(r0, rpt)])

    f = pl.kernel(
        body,
        out_type=jax.ShapeDtypeStruct((2, n_pad, _L), jnp.float32),
        mesh=mesh,
        scratch_types=(
            [pltpu.VMEM_SHARED((n_pad, _L), jnp.float32)]
            + 2 * [pltpu.VMEM((_L,), jnp.int32),
                   pltpu.VMEM((_L,), jnp.int32),
                   pltpu.VMEM((_L, _L), jnp.float32)]
            + 8 * [pltpu.SemaphoreType.DMA]
        ),
    )
    return f(h0, h1, src, dst, zeros_rows)


def _sc_count(dst, ones_blk, zeros_cnt, n):
    e = dst.shape[0]
    nblk = e // _L
    nw = _NSUB * _NCORE
    bpw = -(-nblk // nw)
    rpt = -(-(n // _NSUB) // 8) * 8
    n_pad = rpt * _NSUB
    mesh = plsc.VectorSubcoreMesh(core_axis_name="c", subcore_axis_name="s")

    def body(dst_hbm, ones_hbm, z_hbm, out_hbm, acc, dst_v, ones_v):
        core = lax.axis_index("c")
        sub = lax.axis_index("s")
        wid = sub * _NCORE + core
        r0 = sub * rpt
        pltpu.sync_copy(z_hbm, acc.at[pl.ds(r0, rpt)])
        pltpu.sync_copy(ones_hbm, ones_v)
        plsc.subcore_barrier()

        def blk(j, carry):
            b = wid + nw * j

            @pl.when(b < nblk)
            def _():
                pltpu.sync_copy(dst_hbm.at[pl.ds(b * _L, _L)], dst_v)
                pltpu.sync_copy(ones_v, acc.at[dst_v], add=True)

            return carry

        lax.fori_loop(0, bpw, blk, 0)
        plsc.subcore_barrier()

        @pl.when(core == 0)
        def _():
            pltpu.sync_copy(acc.at[pl.ds(r0, rpt)],
                            out_hbm.at[0, pl.ds(r0, rpt)])

        @pl.when(core == 1)
        def _():
            pltpu.sync_copy(acc.at[pl.ds(r0, rpt)],
                            out_hbm.at[1, pl.ds(r0, rpt)])

    f = pl.kernel(
        body,
        out_type=jax.ShapeDtypeStruct((2, n_pad, _L), jnp.float32),
        mesh=mesh,
        scratch_types=[
            pltpu.VMEM_SHARED((n_pad, _L), jnp.float32),
            pltpu.VMEM((_L,), jnp.int32),
            pltpu.VMEM((_L, _L), jnp.float32),
        ],
    )
    return f(dst, ones_blk, zeros_cnt)


def _tc_layer(acc_list, h_chunks, cnt, Wl, Wr, bias, chunked_out, bn=1000):
    n = h_chunks.shape[1]
    grid_n = n // bn
    n_acc = len(acc_list)
    nc_h = h_chunks.shape[0]
    d_in = Wl.shape[0]
    d_out = Wl.shape[1]
    prec = jax.lax.Precision.HIGHEST

    def body(*refs):
        acc_refs = refs[:n_acc]
        h_ref, cnt_ref, wl_ref, wr_ref, b_ref, o_ref = refs[n_acc:]
        cr = cnt_ref[...]
        tot = cr[0, :, 0:1] + cr[1, :, 0:1]
        inv = 1.0 / jnp.maximum(tot, 1.0)
        wl = wl_ref[...]
        wr = wr_ref[...]
        tmp = jnp.zeros((bn, d_out), jnp.float32)
        ci = 0
        for ar in acc_refs:
            a = ar[...]
            for k in range(a.shape[0]):
                tmp = tmp + lax.dot(a[k], wl[ci * 128:(ci + 1) * 128, :],
                                    precision=prec)
                ci += 1
        tmp = tmp * inv
        h = h_ref[...]
        for k in range(nc_h):
            tmp = tmp + lax.dot(h[k], wr[k * 128:(k + 1) * 128, :],
                                precision=prec)
        out = jnp.maximum(tmp + b_ref[...], 0.0)
        if chunked_out:
            for k in range(d_out // 128):
                o_ref[k] = out[:, k * 128:(k + 1) * 128]
        else:
            o_ref[...] = out

    in_specs = (
        [pl.BlockSpec((2, bn, 128), lambda i: (0, i, 0)) for _ in acc_list]
        + [
            pl.BlockSpec((nc_h, bn, 128), lambda i: (0, i, 0)),
            pl.BlockSpec((2, bn, 128), lambda i: (0, i, 0)),
            pl.BlockSpec((d_in, d_out), lambda i: (0, 0)),
            pl.BlockSpec((d_in, d_out), lambda i: (0, 0)),
            pl.BlockSpec((1, d_out), lambda i: (0, 0)),
        ]
    )
    if chunked_out:
        out_spec = pl.BlockSpec((d_out // 128, bn, 128), lambda i: (0, i, 0))
        out_shape = jax.ShapeDtypeStruct((d_out // 128, n, 128), jnp.float32)
    else:
        out_spec = pl.BlockSpec((bn, d_out), lambda i: (i, 0))
        out_shape = jax.ShapeDtypeStruct((n, d_out), jnp.float32)

    return pl.pallas_call(
        body,
        grid=(grid_n,),
        in_specs=in_specs,
        out_specs=out_spec,
        out_shape=out_shape,
    )(*acc_list, h_chunks, cnt, Wl, Wr, bias)


def kernel(x, edge_index, Wl1, Wr1, b1, Wl2, Wr2, b2, Wl3, Wr3, b3):
    n, d_in = x.shape
    e = edge_index.shape[1]
    src = edge_index[0]
    dst = edge_index[1]
    f32 = jnp.float32

    rpt = -(-(n // _NSUB) // 8) * 8
    xc = x.reshape(n, d_in // 128, 128).transpose(1, 0, 2)
    zrows = jnp.zeros((rpt, _L), f32)
    ones_blk = jnp.ones((_L, _L), f32)

    cnt = _sc_count(dst, ones_blk, zrows, n)

    acc1 = _sc_agg_pair(xc[0], xc[1], src, dst, zrows)
    h1 = _tc_layer([acc1], xc, cnt, Wl1, Wr1, b1.reshape(1, -1), True)

    acc2a = _sc_agg_pair(h1[0], h1[1], src, dst, zrows)
    acc2b = _sc_agg_pair(h1[2], h1[3], src, dst, zrows)
    h2 = _tc_layer([acc2a, acc2b], h1, cnt, Wl2, Wr2, b2.reshape(1, -1), True)

    acc3a = _sc_agg_pair(h2[0], h2[1], src, dst, zrows)
    acc3b = _sc_agg_pair(h2[2], h2[3], src, dst, zrows)
    h3 = _tc_layer([acc3a, acc3b], h2, cnt, Wl3, Wr3, b3.reshape(1, -1), False)
    return h3


# speedup vs baseline: 1.8629x; 1.0137x over previous
import jax
import jax.numpy as jnp
from jax import lax
from jax.experimental import pallas as pl
from jax.experimental.pallas import tpu as pltpu
from jax.experimental.pallas import tpu_sc as plsc

_NSUB = 16
_NCORE = 2
_L = 128


def _sc_agg_pair(h0, h1, src, dst, zeros_rows):
    n = h0.shape[0]
    e = src.shape[0]
    nblk = e // _L
    rpt = -(-(n // _NSUB) // 8) * 8
    n_pad = rpt * _NSUB
    bpt = -(-nblk // _NSUB)
    mesh = plsc.VectorSubcoreMesh(core_axis_name="c", subcore_axis_name="s")

    bpt2 = -(-bpt // 2)

    def body(h0_hbm, h1_hbm, src_hbm, dst_hbm, z_hbm, out_hbm,
             acc, src_v0, dst_v0, msg0, src_v1, dst_v1, msg1,
             gsem0, gsem1, ssem0, ssem1, isem0, isem1, isem2, isem3):
        core = lax.axis_index("c")
        sub = lax.axis_index("s")
        r0 = sub * rpt
        pltpu.sync_copy(z_hbm, acc.at[pl.ds(r0, rpt)])
        plsc.subcore_barrier()

        slots = ((src_v0, dst_v0, msg0, gsem0, ssem0, isem0, isem1),
                 (src_v1, dst_v1, msg1, gsem1, ssem1, isem2, isem3))

        def edge_loop(h_hbm):
            def one_slot(i2, p):
                src_v, dst_v, msg, gsem, ssem, ia, ib = slots[p]
                j = 2 * i2 + p
                b = sub + _NSUB * j

                @pl.when(i2 > 0)
                def _():
                    pltpu.make_async_copy(msg, acc.at[dst_v], ssem).wait()

                @pl.when(b < nblk)
                def _():
                    off = b * _L
                    d1 = pltpu.async_copy(src_hbm.at[pl.ds(off, _L)],
                                          src_v, ia)
                    d2 = pltpu.async_copy(dst_hbm.at[pl.ds(off, _L)],
                                          dst_v, ib)
                    d1.wait()
                    d2.wait()
                    pltpu.async_copy(h_hbm.at[src_v], msg, gsem).wait()
                    pltpu.async_copy(msg, acc.at[dst_v], ssem, add=True)

            def blk2(i2, carry):
                one_slot(i2, 0)
                one_slot(i2, 1)
                return carry

            lax.fori_loop(0, bpt2, blk2, 0)
            for p in range(2):
                src_v, dst_v, msg, gsem, ssem, ia, ib = slots[p]
                last_b = sub + _NSUB * (2 * (bpt2 - 1) + p)

                @pl.when(last_b < nblk)
                def _():
                    pltpu.make_async_copy(msg, acc.at[dst_v], ssem).wait()

        @pl.when(core == 0)
        def _():
            edge_loop(h0_hbm)

        @pl.when(core == 1)
        def _():
            edge_loop(h1_hbm)

        plsc.subcore_barrier()

        @pl.when(core == 0)
        def _():
            pltpu.sync_copy(acc.at[pl.ds(r0, rpt)],
                            out_hbm.at[0, pl.ds(r0, rpt)])

        @pl.when(core == 1)
        def _():
            pltpu.sync_copy(acc.at[pl.ds(r0, rpt)],
                            out_hbm.at[1, pl.ds(r0, rpt)])

    f = pl.kernel(
        body,
        out_type=jax.ShapeDtypeStruct((2, n_pad, _L), jnp.float32),
        mesh=mesh,
        scratch_types=(
            [pltpu.VMEM_SHARED((n_pad, _L), jnp.float32)]
            + 2 * [pltpu.VMEM((_L,), jnp.int32),
                   pltpu.VMEM((_L,), jnp.int32),
                   pltpu.VMEM((_L, _L), jnp.float32)]
            + 8 * [pltpu.SemaphoreType.DMA]
        ),
    )
    return f(h0, h1, src, dst, zeros_rows)


def _sc_count(dst, ones_blk, zeros_cnt, n):
    e = dst.shape[0]
    nblk = e // _L
    nw = _NSUB * _NCORE
    bpw = -(-nblk // nw)
    rpt = -(-(n // _NSUB) // 8) * 8
    n_pad = rpt * _NSUB
    mesh = plsc.VectorSubcoreMesh(core_axis_name="c", subcore_axis_name="s")

    bpw2 = -(-bpw // 2)

    def body(dst_hbm, ones_hbm, z_hbm, out_hbm, acc, ones_v,
             dst_v0, dst_v1, ssem0, ssem1, isem0, isem1):
        core = lax.axis_index("c")
        sub = lax.axis_index("s")
        wid = sub * _NCORE + core
        r0 = sub * rpt
        pltpu.sync_copy(z_hbm, acc.at[pl.ds(r0, rpt)])
        pltpu.sync_copy(ones_hbm, ones_v)
        plsc.subcore_barrier()

        slots = ((dst_v0, ssem0, isem0), (dst_v1, ssem1, isem1))

        def one_slot(i2, p):
            dst_v, ssem, isem = slots[p]
            b = wid + nw * (2 * i2 + p)

            @pl.when(i2 > 0)
            def _():
                pltpu.make_async_copy(ones_v, acc.at[dst_v], ssem).wait()

            @pl.when(b < nblk)
            def _():
                pltpu.async_copy(dst_hbm.at[pl.ds(b * _L, _L)],
                                 dst_v, isem).wait()
                pltpu.async_copy(ones_v, acc.at[dst_v], ssem, add=True)

        def blk2(i2, carry):
            one_slot(i2, 0)
            one_slot(i2, 1)
            return carry

        lax.fori_loop(0, bpw2, blk2, 0)
        for p in range(2):
            dst_v, ssem, isem = slots[p]
            last_b = wid + nw * (2 * (bpw2 - 1) + p)

            @pl.when(last_b < nblk)
            def _():
                pltpu.make_async_copy(ones_v, acc.at[dst_v], ssem).wait()

        plsc.subcore_barrier()

        @pl.when(core == 0)
        def _():
            pltpu.sync_copy(acc.at[pl.ds(r0, rpt)],
                            out_hbm.at[0, pl.ds(r0, rpt)])

        @pl.when(core == 1)
        def _():
            pltpu.sync_copy(acc.at[pl.ds(r0, rpt)],
                            out_hbm.at[1, pl.ds(r0, rpt)])

    f = pl.kernel(
        body,
        out_type=jax.ShapeDtypeStruct((2, n_pad, _L), jnp.float32),
        mesh=mesh,
        scratch_types=(
            [pltpu.VMEM_SHARED((n_pad, _L), jnp.float32),
             pltpu.VMEM((_L, _L), jnp.float32),
             pltpu.VMEM((_L,), jnp.int32),
             pltpu.VMEM((_L,), jnp.int32)]
            + 4 * [pltpu.SemaphoreType.DMA]
        ),
    )
    return f(dst, ones_blk, zeros_cnt)


def _tc_layer(acc_list, h_chunks, cnt, Wl, Wr, bias, chunked_out, bn=1000):
    n = h_chunks.shape[1]
    grid_n = n // bn
    n_acc = len(acc_list)
    nc_h = h_chunks.shape[0]
    d_in = Wl.shape[0]
    d_out = Wl.shape[1]
    prec = jax.lax.Precision.HIGHEST

    def body(*refs):
        acc_refs = refs[:n_acc]
        h_ref, cnt_ref, wl_ref, wr_ref, b_ref, o_ref = refs[n_acc:]
        cr = cnt_ref[...]
        tot = cr[0, :, 0:1] + cr[1, :, 0:1]
        inv = 1.0 / jnp.maximum(tot, 1.0)
        wl = wl_ref[...]
        wr = wr_ref[...]
        tmp = jnp.zeros((bn, d_out), jnp.float32)
        ci = 0
        for ar in acc_refs:
            a = ar[...]
            for k in range(a.shape[0]):
                tmp = tmp + lax.dot(a[k], wl[ci * 128:(ci + 1) * 128, :],
                                    precision=prec)
                ci += 1
        tmp = tmp * inv
        h = h_ref[...]
        for k in range(nc_h):
            tmp = tmp + lax.dot(h[k], wr[k * 128:(k + 1) * 128, :],
                                precision=prec)
        out = jnp.maximum(tmp + b_ref[...], 0.0)
        if chunked_out:
            for k in range(d_out // 128):
                o_ref[k] = out[:, k * 128:(k + 1) * 128]
        else:
            o_ref[...] = out

    in_specs = (
        [pl.BlockSpec((2, bn, 128), lambda i: (0, i, 0)) for _ in acc_list]
        + [
            pl.BlockSpec((nc_h, bn, 128), lambda i: (0, i, 0)),
            pl.BlockSpec((2, bn, 128), lambda i: (0, i, 0)),
            pl.BlockSpec((d_in, d_out), lambda i: (0, 0)),
            pl.BlockSpec((d_in, d_out), lambda i: (0, 0)),
            pl.BlockSpec((1, d_out), lambda i: (0, 0)),
        ]
    )
    if chunked_out:
        out_spec = pl.BlockSpec((d_out // 128, bn, 128), lambda i: (0, i, 0))
        out_shape = jax.ShapeDtypeStruct((d_out // 128, n, 128), jnp.float32)
    else:
        out_spec = pl.BlockSpec((bn, d_out), lambda i: (i, 0))
        out_shape = jax.ShapeDtypeStruct((n, d_out), jnp.float32)

    return pl.pallas_call(
        body,
        grid=(grid_n,),
        in_specs=in_specs,
        out_specs=out_spec,
        out_shape=out_shape,
    )(*acc_list, h_chunks, cnt, Wl, Wr, bias)


def kernel(x, edge_index, Wl1, Wr1, b1, Wl2, Wr2, b2, Wl3, Wr3, b3):
    n, d_in = x.shape
    e = edge_index.shape[1]
    src = edge_index[0]
    dst = edge_index[1]
    f32 = jnp.float32

    rpt = -(-(n // _NSUB) // 8) * 8
    xc = x.reshape(n, d_in // 128, 128).transpose(1, 0, 2)
    zrows = jnp.zeros((rpt, _L), f32)
    ones_blk = jnp.ones((_L, _L), f32)

    cnt = _sc_count(dst, ones_blk, zrows, n)

    acc1 = _sc_agg_pair(xc[0], xc[1], src, dst, zrows)
    h1 = _tc_layer([acc1], xc, cnt, Wl1, Wr1, b1.reshape(1, -1), True)

    acc2a = _sc_agg_pair(h1[0], h1[1], src, dst, zrows)
    acc2b = _sc_agg_pair(h1[2], h1[3], src, dst, zrows)
    h2 = _tc_layer([acc2a, acc2b], h1, cnt, Wl2, Wr2, b2.reshape(1, -1), True)

    acc3a = _sc_agg_pair(h2[0], h2[1], src, dst, zrows)
    acc3b = _sc_agg_pair(h2[2], h2[3], src, dst, zrows)
    h3 = _tc_layer([acc3a, acc3b], h2, cnt, Wl3, Wr3, b3.reshape(1, -1), False)
    return h3
